# Initial kernel scaffold; baseline (speedup 1.0000x reference)
import jax
import jax.numpy as jnp
from jax.experimental import pallas as pl


def kernel(h, adj_indices, W, a):
    raise NotImplementedError("write your pallas kernel here")


# speedup vs baseline: 13.0682x; 13.0682x over previous
import functools

import jax
import jax.numpy as jnp
from jax import lax
from jax.experimental import pallas as pl
from jax.experimental.pallas import tpu as pltpu
from jax.experimental.pallas import tpu_sc as plsc

ALPHA = 0.2
NC = 2
NS = 16
NW = NC * NS
CH = 128
PAD = 16


def _proj_body(h_ref, w_ref, a_ref, hp_ref, as_ref, ad_ref):
    d = w_ref.shape[0]
    rb = h_ref.shape[0]
    hp = jnp.dot(h_ref[...], w_ref[...], preferred_element_type=jnp.float32)
    lane = lax.broadcasted_iota(jnp.int32, (rb, PAD), 1)
    ones = jnp.where(lane == 0, 1.0, 0.0).astype(jnp.float32)
    hp_ref[...] = jnp.concatenate([hp, ones], axis=1)
    as_ref[...] = jnp.dot(hp, a_ref[:d, :], preferred_element_type=jnp.float32)
    ad_ref[...] = jnp.dot(hp, a_ref[d:, :], preferred_element_type=jnp.float32)


def _make_proj(n, d, rb):
    return pl.pallas_call(
        _proj_body,
        grid=(n // rb,),
        in_specs=[
            pl.BlockSpec((rb, d), lambda i: (i, 0)),
            pl.BlockSpec((d, d), lambda i: (0, 0)),
            pl.BlockSpec((2 * d, 1), lambda i: (0, 0)),
        ],
        out_specs=[
            pl.BlockSpec((rb, d + PAD), lambda i: (i, 0)),
            pl.BlockSpec((rb, 1), lambda i: (i, 0)),
            pl.BlockSpec((rb, 1), lambda i: (i, 0)),
        ],
        out_shape=[
            jax.ShapeDtypeStruct((n, d + PAD), jnp.float32),
            jax.ShapeDtypeStruct((n, 1), jnp.float32),
            jax.ShapeDtypeStruct((n, 1), jnp.float32),
        ],
    )


def _make_edge(n, e, d):
    da = d + PAD
    epw = e // NW
    nfull = epw // CH
    tail = epw - nfull * CH
    assert tail % 16 == 0
    rpt = n // NS
    zblk = rpt // 5
    mesh = plsc.VectorSubcoreMesh(core_axis_name="c", subcore_axis_name="s")

    @functools.partial(
        pl.kernel,
        out_type=jax.ShapeDtypeStruct((NC, n, da), jnp.float32),
        mesh=mesh,
        scratch_types=[
            pltpu.VMEM((n,), jnp.float32),
            pltpu.VMEM((n,), jnp.float32),
            pltpu.VMEM((CH,), jnp.int32),
            pltpu.VMEM((CH,), jnp.int32),
            pltpu.VMEM((CH,), jnp.float32),
            pltpu.VMEM((CH, da), jnp.float32),
            pltpu.VMEM_SHARED((n, da), jnp.float32),
            pltpu.SemaphoreType.DMA,
        ],
        compiler_params=pltpu.CompilerParams(
            use_tc_tiling_on_sc=False, needs_layout_passes=False),
    )
    def edge_kernel(asrc_hbm, adst_hbm, src_hbm, dst_hbm, hp_hbm, out_hbm,
                    asrc_v, adst_v, sidx, didx, e_chunk, rows, acc, gsem):
        c = lax.axis_index("c")
        s = lax.axis_index("s")
        wid = c * NS + s
        ebase = wid * epw
        row0 = s * rpt
        zf = jnp.zeros((16,), jnp.float32)
        zi = jnp.zeros((16,), jnp.int32)

        pltpu.sync_copy(asrc_hbm, asrc_v)
        pltpu.sync_copy(adst_hbm, adst_v)

        def _zrow(i, carry):
            for r in range(da // 16):
                rows[i, pl.ds(r * 16, 16)] = zf
            return carry
        lax.fori_loop(0, CH, _zrow, 0)
        for j in range(5):
            pltpu.sync_copy(rows.at[pl.ds(0, zblk), :],
                            acc.at[pl.ds(row0 + j * zblk, zblk), :])

        def _mx(i, m):
            return jnp.maximum(m, asrc_v[pl.ds(i * 16, 16)])
        ms = jnp.max(lax.fori_loop(0, n // 16, _mx,
                                   jnp.full((16,), -jnp.inf, jnp.float32)))
        def _mx2(i, m):
            return jnp.maximum(m, adst_v[pl.ds(i * 16, 16)])
        md = jnp.max(lax.fori_loop(0, n // 16, _mx2,
                                   jnp.full((16,), -jnp.inf, jnp.float32)))
        raw = ms + md
        shift = jnp.where(raw >= 0.0, raw, ALPHA * raw)

        plsc.subcore_barrier()

        def _e_group(g, carry):
            si = sidx[pl.ds(g * 16, 16)]
            di = didx[pl.ds(g * 16, 16)]
            av = plsc.load_gather(asrc_v, [si])
            bv = plsc.load_gather(adst_v, [di])
            att = av + bv
            att = jnp.where(att >= 0.0, att, ALPHA * att)
            e_chunk[pl.ds(g * 16, 16)] = jnp.exp(att - shift)
            return carry

        def _scale_edge(i, carry):
            ev = plsc.load_gather(e_chunk, [jnp.full((16,), i, jnp.int32)])
            for r in range(da // 16):
                rows[i, pl.ds(r * 16, 16)] = rows[i, pl.ds(r * 16, 16)] * ev
            return carry

        def _do_chunk(k, nedge):
            pltpu.sync_copy(src_hbm.at[pl.ds(ebase + k * CH, nedge)],
                            sidx.at[pl.ds(0, nedge)])
            pltpu.sync_copy(dst_hbm.at[pl.ds(ebase + k * CH, nedge)],
                            didx.at[pl.ds(0, nedge)])
            cp = pltpu.async_copy(hp_hbm.at[sidx], rows, gsem)
            lax.fori_loop(0, nedge // 16, _e_group, 0)
            cp.wait()
            lax.fori_loop(0, CH, _scale_edge, 0)
            pltpu.sync_copy(rows, acc.at[didx], add=True)

        def _chunk(k, carry):
            _do_chunk(k, CH)
            return carry
        lax.fori_loop(0, nfull, _chunk, 0)

        if tail:
            for r in range(CH // 16):
                sidx[pl.ds(r * 16, 16)] = zi
                didx[pl.ds(r * 16, 16)] = zi
                e_chunk[pl.ds(r * 16, 16)] = zf
            _do_chunk(nfull, tail)

        plsc.subcore_barrier()

        pltpu.sync_copy(acc.at[pl.ds(row0, rpt), :],
                        out_hbm.at[c, pl.ds(row0, rpt), :])

    return edge_kernel


def _comb_body(p_ref, o_ref):
    d = o_ref.shape[-1]
    ssum = p_ref[0] + p_ref[1]
    num = ssum[:, :d]
    den = ssum[:, d:d + 1]
    o_ref[...] = jnp.where(den > 0.0, num / den, 0.0)


def _make_comb(n, d, rb):
    return pl.pallas_call(
        _comb_body,
        grid=(n // rb,),
        in_specs=[pl.BlockSpec((NC, rb, d + PAD), lambda i: (0, i, 0))],
        out_specs=pl.BlockSpec((rb, d), lambda i: (i, 0)),
        out_shape=jax.ShapeDtypeStruct((n, d), jnp.float32),
    )


def kernel(h, adj_indices, W, a):
    n, d = h.shape
    e = adj_indices.shape[1]
    src = adj_indices[0]
    dst = adj_indices[1]

    hp, as2, ad2 = _make_proj(n, d, 2000)(h, W, a)
    asrc = as2.reshape(n)
    adst = ad2.reshape(n)

    parts = _make_edge(n, e, d)(asrc, adst, src, dst, hp)
    out = _make_comb(n, d, 2000)(parts)
    return out
